# baseline (speedup 1.0000x reference)
import functools

import jax
import jax.numpy as jnp
from jax import lax
from jax.experimental import pallas as pl
from jax.experimental.pallas import tpu as pltpu
from jax.experimental.pallas import tpu_sc as plsc

_NUM_EXPERTS = 8
_EPG = 4
_ROUTE_SCALE = 2.5
_NEG_INF = float("-inf")


def _score_kernel(x_ref, w_ref, st_ref):
    st = lax.dot_general(
        w_ref[...], x_ref[...],
        dimension_numbers=(((0,), (1,)), ((), ())),
        preferred_element_type=jnp.float32,
    )
    st_ref[...] = jax.nn.sigmoid(st)


def _f16(v, dt=jnp.float32):
    return jnp.full((16,), v, dtype=dt)


def _make_route_kernel(num_tokens):
    info = plsc.get_sparse_core_info()
    nc, ns = info.num_cores, info.num_subcores
    nw = nc * ns
    tok_w = num_tokens // nw
    mesh = plsc.VectorSubcoreMesh(core_axis_name="c", subcore_axis_name="s")

    @functools.partial(
        pl.kernel, mesh=mesh,
        out_type=[
            jax.ShapeDtypeStruct((2, num_tokens), jnp.float32),
            jax.ShapeDtypeStruct((2, num_tokens), jnp.int32),
        ],
        scratch_types=[
            pltpu.VMEM((_NUM_EXPERTS, tok_w), jnp.float32),
            pltpu.VMEM((2, tok_w), jnp.float32),
            pltpu.VMEM((2, tok_w), jnp.int32),
        ],
    )
    def _route(scores_hbm, wout_hbm, iout_hbm, s_v, w_v, i_v):
        wid = lax.axis_index("s") * nc + lax.axis_index("c")
        base = wid * tok_w
        pltpu.sync_copy(scores_hbm.at[:, pl.ds(base, tok_w)], s_v)

        def body(chunk, _):
            sl = pl.ds(chunk * 16, 16)
            s = [s_v[e, sl] for e in range(_NUM_EXPERTS)]
            gmax0 = jnp.maximum(jnp.maximum(s[0], s[1]), jnp.maximum(s[2], s[3]))
            gmax1 = jnp.maximum(jnp.maximum(s[4], s[5]), jnp.maximum(s[6], s[7]))
            sel1 = gmax1 > gmax0
            c = [jnp.where(sel1, s[e + _EPG], s[e]) for e in range(_EPG)]
            m1 = jnp.maximum(jnp.maximum(c[0], c[1]), jnp.maximum(c[2], c[3]))
            i1 = jnp.where(
                c[0] == m1, _f16(0, jnp.int32),
                jnp.where(c[1] == m1, _f16(1, jnp.int32),
                          jnp.where(c[2] == m1, _f16(2, jnp.int32),
                                    _f16(3, jnp.int32))))
            d = [jnp.where(i1 == _f16(e, jnp.int32), _f16(_NEG_INF), c[e])
                 for e in range(_EPG)]
            m2 = jnp.maximum(jnp.maximum(d[0], d[1]), jnp.maximum(d[2], d[3]))
            i2 = jnp.where(
                d[0] == m2, _f16(0, jnp.int32),
                jnp.where(d[1] == m2, _f16(1, jnp.int32),
                          jnp.where(d[2] == m2, _f16(2, jnp.int32),
                                    _f16(3, jnp.int32))))
            goff = jnp.where(sel1, _f16(_EPG, jnp.int32), _f16(0, jnp.int32))
            inv = _f16(_ROUTE_SCALE) / (m1 + m2)
            w_v[0, sl] = m1 * inv
            w_v[1, sl] = m2 * inv
            i_v[0, sl] = i1 + goff
            i_v[1, sl] = i2 + goff
            return ()

        lax.fori_loop(0, tok_w // 16, body, ())
        pltpu.sync_copy(w_v, wout_hbm.at[:, pl.ds(base, tok_w)])
        pltpu.sync_copy(i_v, iout_hbm.at[:, pl.ds(base, tok_w)])

    return _route


@jax.jit
def kernel(x, W):
    num_tokens, hidden = x.shape
    block_t = 4096
    scores = pl.pallas_call(
        _score_kernel,
        grid=(num_tokens // block_t,),
        in_specs=[
            pl.BlockSpec((block_t, hidden), lambda i: (i, 0)),
            pl.BlockSpec((hidden, _NUM_EXPERTS), lambda i: (0, 0)),
        ],
        out_specs=pl.BlockSpec((_NUM_EXPERTS, block_t), lambda i: (0, i)),
        out_shape=jax.ShapeDtypeStruct((_NUM_EXPERTS, num_tokens), jnp.float32),
        compiler_params=pltpu.CompilerParams(
            dimension_semantics=("parallel",),
        ),
    )(x, W)
    weights_t, idx_t = _make_route_kernel(num_tokens)(scores)
    return weights_t.T.astype(x.dtype), idx_t.T
